# baseline (speedup 1.0000x reference)
import jax
import jax.numpy as jnp
import numpy as np
from jax.experimental import pallas as pl

_N = 4096
_FIN = 256
_HID = 64
_HEADS = 4
_NC = 128
_LD = 256

_RB1 = 512
_RB = 512
_RB2 = 512


def _prep1_kernel(x_ref, layout_ref, W1_ref, Msrc_ref, Mdst_ref, Wl_ref,
                  bl_ref, h1e_ref, fsrc_ref, fdstT_ref, lemb_ref):
    h1 = jnp.dot(x_ref[:], W1_ref[:], preferred_element_type=jnp.float32)
    rb = h1.shape[0]
    fd = jnp.dot(h1, Mdst_ref[:], preferred_element_type=jnp.float32)
    Bc = jnp.exp(fd)
    parts = []
    for h in range(_HEADS):
        bh = Bc[:, h:h + 1]
        parts += [h1[:, _HID * h:_HID * (h + 1)] * bh, bh,
                  jnp.zeros((rb, 128 - _HID - 1), jnp.float32)]
    h1e_ref[:] = jnp.concatenate(parts, axis=1).astype(jnp.bfloat16)
    fsrc_ref[:] = jnp.dot(h1, Msrc_ref[:], preferred_element_type=jnp.float32)
    fdT = jax.lax.dot_general(Mdst_ref[:], h1, (((0,), (1,)), ((), ())),
                              preferred_element_type=jnp.float32)
    fdstT_ref[:] = jnp.concatenate(
        [jnp.exp(-0.8 * fdT),
         jnp.zeros((4, fdT.shape[1]), jnp.float32)], axis=0).astype(jnp.bfloat16)
    lemb_ref[:] = jnp.maximum(
        jnp.dot(layout_ref[:], Wl_ref[:], preferred_element_type=jnp.float32)
        + bl_ref[:], 0.0)


def _attn1_kernel(a_ref, fsrc_ref, fdstT_ref, h1e_ref, lemb_ref, b1_ref,
                  W2_ref, asrc2_ref, adst2_ref, Wp_ref,
                  h2e_ref, f2src_ref, f2dstT_ref, pk_ref):
    abf = a_ref[:].astype(jnp.bfloat16)
    P = jax.lax.dot_general(Wp_ref[:], abf, (((1,), (0,)), ((), ())),
                            preferred_element_type=jnp.float32)
    pk_ref[:] = P.astype(jnp.int32)
    one = jnp.bfloat16(1.0)
    outs = []
    for h in range(_HEADS):
        E = jnp.exp(-0.8 * fsrc_ref[:, h:h + 1]).astype(jnp.bfloat16)
        F = fdstT_ref[h:h + 1, :]
        p = jnp.maximum(E * F, one) * abf
        agg = jnp.dot(p, h1e_ref[:, 128 * h:128 * (h + 1)],
                      preferred_element_type=jnp.float32)
        oh = agg[:, :_HID] / agg[:, _HID:_HID + 1]
        outs.append(oh)
    o = jnp.concatenate(outs, axis=1) + b1_ref[:]
    o = jnp.where(o > 0, o, jnp.exp(jnp.minimum(o, 0.0)) - 1.0)
    x1g = o + lemb_ref[:]
    h2 = jnp.dot(x1g, W2_ref[:], preferred_element_type=jnp.float32)
    rb = h2.shape[0]
    f2d = jnp.dot(h2, adst2_ref[:], preferred_element_type=jnp.float32)
    B2 = jnp.exp(f2d)
    h2e_ref[:] = jnp.concatenate(
        [h2 * B2, B2, jnp.zeros((rb, 127), jnp.float32)],
        axis=1).astype(jnp.bfloat16)
    f2src_ref[:] = jnp.dot(h2, asrc2_ref[:], preferred_element_type=jnp.float32)
    fdT2 = jax.lax.dot_general(adst2_ref[:], h2, (((0,), (1,)), ((), ())),
                               preferred_element_type=jnp.float32)
    f2dstT_ref[:] = jnp.concatenate(
        [jnp.exp(-0.8 * fdT2),
         jnp.zeros((7, fdT2.shape[1]), jnp.float32)],
        axis=0).astype(jnp.bfloat16)


def _attn2_kernel(pk_ref, f2src_ref, f2dstT_ref, h2e_ref, b2_ref, out_ref):
    E = jnp.exp(-0.8 * f2src_ref[:]).astype(jnp.bfloat16)
    F = f2dstT_ref[0:1, :]
    one = jnp.bfloat16(1.0)
    pk = pk_ref[:]
    h2e = h2e_ref[:]
    aggs = []
    for k in range(_RB2 // 32):
        bits = ((pk >> k) & 1).astype(jnp.bfloat16)
        Ek = E[32 * k:32 * (k + 1), :]
        p = jnp.maximum(Ek * F, one) * bits
        aggs.append(jnp.dot(p, h2e, preferred_element_type=jnp.float32))
    agg = jnp.concatenate(aggs, axis=0)
    z = agg[:, :_NC] / agg[:, _NC:_NC + 1] + b2_ref[:]
    m2 = jnp.max(z, axis=1, keepdims=True)
    q = jnp.exp(z - m2)
    out_ref[:] = q / jnp.sum(q, axis=1, keepdims=True)


def kernel(x, a, layout, W1, asrc1, adst1, b1, Wl, bl, W2, asrc2, adst2, b2):
    f32 = jnp.float32
    W1f = W1.reshape(_FIN, _HEADS * _HID)
    sel = jnp.asarray(np.repeat(np.eye(_HEADS, dtype=np.float32), _HID, axis=0))
    Msrc = asrc1.reshape(-1)[:, None] * sel
    Mdst = adst1.reshape(-1)[:, None] * sel
    bl2 = bl.reshape(1, -1)
    b12 = b1.reshape(1, -1)
    b22 = b2.reshape(1, -1)
    W2f = W2.reshape(_HEADS * _HID, _NC)
    asrc2c = asrc2.reshape(_NC, 1)
    adst2c = adst2.reshape(_NC, 1)
    Wp_np = np.zeros((32, _RB), dtype=np.float32)
    for l in range(_RB):
        Wp_np[l % 32, l] = float(2 ** (l // 32))
    Wp = jnp.asarray(Wp_np, dtype=jnp.bfloat16)

    nb1 = _N // _RB1
    h1e, fsrc, fdstT, lemb = pl.pallas_call(
        _prep1_kernel,
        grid=(nb1,),
        in_specs=[
            pl.BlockSpec((_RB1, _FIN), lambda i: (i, 0)),
            pl.BlockSpec((_RB1, _LD), lambda i: (i, 0)),
            pl.BlockSpec((_FIN, _HEADS * _HID), lambda i: (0, 0)),
            pl.BlockSpec((_FIN, _HEADS), lambda i: (0, 0)),
            pl.BlockSpec((_FIN, _HEADS), lambda i: (0, 0)),
            pl.BlockSpec((_LD, _HEADS * _HID), lambda i: (0, 0)),
            pl.BlockSpec((1, _HEADS * _HID), lambda i: (0, 0)),
        ],
        out_specs=[
            pl.BlockSpec((_RB1, _HEADS * 128), lambda i: (i, 0)),
            pl.BlockSpec((_RB1, _HEADS), lambda i: (i, 0)),
            pl.BlockSpec((8, _RB1), lambda i: (0, i)),
            pl.BlockSpec((_RB1, _HEADS * _HID), lambda i: (i, 0)),
        ],
        out_shape=[
            jax.ShapeDtypeStruct((_N, _HEADS * 128), jnp.bfloat16),
            jax.ShapeDtypeStruct((_N, _HEADS), f32),
            jax.ShapeDtypeStruct((8, _N), jnp.bfloat16),
            jax.ShapeDtypeStruct((_N, _HEADS * _HID), f32),
        ],
    )(x, layout, W1f, Msrc, Mdst, Wl, bl2)

    nb = _N // _RB
    h2e, f2src, f2dstT, pk = pl.pallas_call(
        _attn1_kernel,
        grid=(nb,),
        in_specs=[
            pl.BlockSpec((_RB, _N), lambda i: (i, 0)),
            pl.BlockSpec((_RB, _HEADS), lambda i: (i, 0)),
            pl.BlockSpec((8, _N), lambda i: (0, 0)),
            pl.BlockSpec((_N, _HEADS * 128), lambda i: (0, 0)),
            pl.BlockSpec((_RB, _HEADS * _HID), lambda i: (i, 0)),
            pl.BlockSpec((1, _HEADS * _HID), lambda i: (0, 0)),
            pl.BlockSpec((_HEADS * _HID, _NC), lambda i: (0, 0)),
            pl.BlockSpec((_NC, 1), lambda i: (0, 0)),
            pl.BlockSpec((_NC, 1), lambda i: (0, 0)),
            pl.BlockSpec((32, _RB), lambda i: (0, 0)),
        ],
        out_specs=[
            pl.BlockSpec((_RB, 2 * _NC), lambda i: (i, 0)),
            pl.BlockSpec((_RB, 1), lambda i: (i, 0)),
            pl.BlockSpec((8, _RB), lambda i: (0, i)),
            pl.BlockSpec((32, _N), lambda i: (i, 0)),
        ],
        out_shape=[
            jax.ShapeDtypeStruct((_N, 2 * _NC), jnp.bfloat16),
            jax.ShapeDtypeStruct((_N, 1), f32),
            jax.ShapeDtypeStruct((8, _N), jnp.bfloat16),
            jax.ShapeDtypeStruct((32 * (_N // _RB), _N), jnp.int32),
        ],
    )(a, fsrc, fdstT, h1e, lemb, b12, W2f, asrc2c, adst2c, Wp)

    out = pl.pallas_call(
        _attn2_kernel,
        grid=(_N // _RB2,),
        in_specs=[
            pl.BlockSpec((32, _N), lambda i: (i, 0)),
            pl.BlockSpec((_RB2, 1), lambda i: (i, 0)),
            pl.BlockSpec((8, _N), lambda i: (0, 0)),
            pl.BlockSpec((_N, 2 * _NC), lambda i: (0, 0)),
            pl.BlockSpec((1, _NC), lambda i: (0, 0)),
        ],
        out_specs=pl.BlockSpec((_RB2, _NC), lambda i: (i, 0)),
        out_shape=jax.ShapeDtypeStruct((_N, _NC), f32),
    )(pk, f2src, f2dstT, h2e, b22)
    return out


# speedup vs baseline: 1.3799x; 1.3799x over previous
import jax
import jax.numpy as jnp
import numpy as np
from jax.experimental import pallas as pl

_N = 4096
_FIN = 256
_HID = 64
_HEADS = 4
_NC = 128
_LD = 256

_RB1 = 512
_RB = 512
_RB2 = 1024


def _prep1_kernel(x_ref, layout_ref, W1_ref, Msrc_ref, Mdst_ref, Wl_ref,
                  bl_ref, h1e_ref, fsrc_ref, fdstT_ref, lemb_ref):
    h1 = jnp.dot(x_ref[:], W1_ref[:], preferred_element_type=jnp.float32)
    rb = h1.shape[0]
    fd = jnp.dot(h1, Mdst_ref[:], preferred_element_type=jnp.float32)
    Bc = jnp.exp(fd)
    parts = []
    for h in range(_HEADS):
        bh = Bc[:, h:h + 1]
        parts += [h1[:, _HID * h:_HID * (h + 1)] * bh, bh,
                  jnp.zeros((rb, 128 - _HID - 1), jnp.float32)]
    h1e_ref[:] = jnp.concatenate(parts, axis=1).astype(jnp.bfloat16)
    fsrc_ref[:] = jnp.dot(h1, Msrc_ref[:], preferred_element_type=jnp.float32)
    fdT = jax.lax.dot_general(Mdst_ref[:], h1, (((0,), (1,)), ((), ())),
                              preferred_element_type=jnp.float32)
    fdstT_ref[:] = jnp.concatenate(
        [jnp.exp(-0.8 * fdT),
         jnp.zeros((4, fdT.shape[1]), jnp.float32)], axis=0).astype(jnp.bfloat16)
    lemb_ref[:] = jnp.maximum(
        jnp.dot(layout_ref[:], Wl_ref[:], preferred_element_type=jnp.float32)
        + bl_ref[:], 0.0)


def _attn1_kernel(a_ref, fsrc_ref, fdstT_ref, h1e_ref, lemb_ref, b1_ref,
                  W2_ref, asrc2_ref, adst2_ref,
                  h2e_ref, f2src_ref, f2dstT_ref, abf_ref):
    abf = a_ref[:].astype(jnp.bfloat16)
    abf_ref[:] = abf
    one = jnp.bfloat16(1.0)
    outs = []
    for h in range(_HEADS):
        E = jnp.exp(-0.8 * fsrc_ref[:, h:h + 1]).astype(jnp.bfloat16)
        F = fdstT_ref[h:h + 1, :]
        p = jnp.maximum(E * F, one) * abf
        agg = jnp.dot(p, h1e_ref[:, 128 * h:128 * (h + 1)],
                      preferred_element_type=jnp.float32)
        oh = agg[:, :_HID] / agg[:, _HID:_HID + 1]
        outs.append(oh)
    o = jnp.concatenate(outs, axis=1) + b1_ref[:]
    o = jnp.where(o > 0, o, jnp.exp(jnp.minimum(o, 0.0)) - 1.0)
    x1g = o + lemb_ref[:]
    h2 = jnp.dot(x1g, W2_ref[:], preferred_element_type=jnp.float32)
    rb = h2.shape[0]
    f2d = jnp.dot(h2, adst2_ref[:], preferred_element_type=jnp.float32)
    B2 = jnp.exp(f2d)
    h2e_ref[:] = jnp.concatenate(
        [h2 * B2, B2, jnp.zeros((rb, 127), jnp.float32)],
        axis=1).astype(jnp.bfloat16)
    f2src_ref[:] = jnp.dot(h2, asrc2_ref[:], preferred_element_type=jnp.float32)
    fdT2 = jax.lax.dot_general(adst2_ref[:], h2, (((0,), (1,)), ((), ())),
                               preferred_element_type=jnp.float32)
    f2dstT_ref[:] = jnp.concatenate(
        [jnp.exp(-0.8 * fdT2),
         jnp.zeros((7, fdT2.shape[1]), jnp.float32)],
        axis=0).astype(jnp.bfloat16)


def _attn2_kernel(abf_ref, f2src_ref, f2dstT_ref, h2e_ref, b2_ref, out_ref):
    E = jnp.exp(-0.8 * f2src_ref[:]).astype(jnp.bfloat16)
    F = f2dstT_ref[0:1, :]
    p = jnp.maximum(E * F, jnp.bfloat16(1.0)) * abf_ref[:]
    agg = jnp.dot(p, h2e_ref[:], preferred_element_type=jnp.float32)
    z = agg[:, :_NC] / agg[:, _NC:_NC + 1] + b2_ref[:]
    m2 = jnp.max(z, axis=1, keepdims=True)
    q = jnp.exp(z - m2)
    out_ref[:] = q / jnp.sum(q, axis=1, keepdims=True)


def kernel(x, a, layout, W1, asrc1, adst1, b1, Wl, bl, W2, asrc2, adst2, b2):
    f32 = jnp.float32
    W1f = W1.reshape(_FIN, _HEADS * _HID)
    sel = jnp.asarray(np.repeat(np.eye(_HEADS, dtype=np.float32), _HID, axis=0))
    Msrc = asrc1.reshape(-1)[:, None] * sel
    Mdst = adst1.reshape(-1)[:, None] * sel
    bl2 = bl.reshape(1, -1)
    b12 = b1.reshape(1, -1)
    b22 = b2.reshape(1, -1)
    W2f = W2.reshape(_HEADS * _HID, _NC)
    asrc2c = asrc2.reshape(_NC, 1)
    adst2c = adst2.reshape(_NC, 1)

    nb1 = _N // _RB1
    h1e, fsrc, fdstT, lemb = pl.pallas_call(
        _prep1_kernel,
        grid=(nb1,),
        in_specs=[
            pl.BlockSpec((_RB1, _FIN), lambda i: (i, 0)),
            pl.BlockSpec((_RB1, _LD), lambda i: (i, 0)),
            pl.BlockSpec((_FIN, _HEADS * _HID), lambda i: (0, 0)),
            pl.BlockSpec((_FIN, _HEADS), lambda i: (0, 0)),
            pl.BlockSpec((_FIN, _HEADS), lambda i: (0, 0)),
            pl.BlockSpec((_LD, _HEADS * _HID), lambda i: (0, 0)),
            pl.BlockSpec((1, _HEADS * _HID), lambda i: (0, 0)),
        ],
        out_specs=[
            pl.BlockSpec((_RB1, _HEADS * 128), lambda i: (i, 0)),
            pl.BlockSpec((_RB1, _HEADS), lambda i: (i, 0)),
            pl.BlockSpec((8, _RB1), lambda i: (0, i)),
            pl.BlockSpec((_RB1, _HEADS * _HID), lambda i: (i, 0)),
        ],
        out_shape=[
            jax.ShapeDtypeStruct((_N, _HEADS * 128), jnp.bfloat16),
            jax.ShapeDtypeStruct((_N, _HEADS), f32),
            jax.ShapeDtypeStruct((8, _N), jnp.bfloat16),
            jax.ShapeDtypeStruct((_N, _HEADS * _HID), f32),
        ],
    )(x, layout, W1f, Msrc, Mdst, Wl, bl2)

    nb = _N // _RB
    h2e, f2src, f2dstT, abf = pl.pallas_call(
        _attn1_kernel,
        grid=(nb,),
        in_specs=[
            pl.BlockSpec((_RB, _N), lambda i: (i, 0)),
            pl.BlockSpec((_RB, _HEADS), lambda i: (i, 0)),
            pl.BlockSpec((8, _N), lambda i: (0, 0)),
            pl.BlockSpec((_N, _HEADS * 128), lambda i: (0, 0)),
            pl.BlockSpec((_RB, _HEADS * _HID), lambda i: (i, 0)),
            pl.BlockSpec((1, _HEADS * _HID), lambda i: (0, 0)),
            pl.BlockSpec((_HEADS * _HID, _NC), lambda i: (0, 0)),
            pl.BlockSpec((_NC, 1), lambda i: (0, 0)),
            pl.BlockSpec((_NC, 1), lambda i: (0, 0)),
        ],
        out_specs=[
            pl.BlockSpec((_RB, 2 * _NC), lambda i: (i, 0)),
            pl.BlockSpec((_RB, 1), lambda i: (i, 0)),
            pl.BlockSpec((8, _RB), lambda i: (0, i)),
            pl.BlockSpec((_RB, _N), lambda i: (i, 0)),
        ],
        out_shape=[
            jax.ShapeDtypeStruct((_N, 2 * _NC), jnp.bfloat16),
            jax.ShapeDtypeStruct((_N, 1), f32),
            jax.ShapeDtypeStruct((8, _N), jnp.bfloat16),
            jax.ShapeDtypeStruct((_N, _N), jnp.bfloat16),
        ],
    )(a, fsrc, fdstT, h1e, lemb, b12, W2f, asrc2c, adst2c)

    out = pl.pallas_call(
        _attn2_kernel,
        grid=(_N // _RB2,),
        in_specs=[
            pl.BlockSpec((_RB2, _N), lambda i: (i, 0)),
            pl.BlockSpec((_RB2, 1), lambda i: (i, 0)),
            pl.BlockSpec((8, _N), lambda i: (0, 0)),
            pl.BlockSpec((_N, 2 * _NC), lambda i: (0, 0)),
            pl.BlockSpec((1, _NC), lambda i: (0, 0)),
        ],
        out_specs=pl.BlockSpec((_RB2, _NC), lambda i: (i, 0)),
        out_shape=jax.ShapeDtypeStruct((_N, _NC), f32),
    )(abf, f2src, f2dstT, h2e, b22)
    return out
